# Initial kernel scaffold; baseline (speedup 1.0000x reference)
import jax
import jax.numpy as jnp
from jax.experimental import pallas as pl


def kernel(x, edge_index, edge_attr, batch, W1, b1, W2, b2):
    raise NotImplementedError("write your pallas kernel here")


# speedup vs baseline: 10.7085x; 10.7085x over previous
import functools

import jax
import jax.numpy as jnp
from jax import lax
from jax.experimental import pallas as pl
from jax.experimental.pallas import tpu as pltpu
from jax.experimental.pallas import tpu_sc as plsc

N = 10000
E = 160000
D = 256
H = 64
G = 128

NC = 2
NS = 16
NW = NC * NS
CHUNK = 128
CPT = 40
E_PAD = NW * CPT * CHUNK
N_PAD = 10240
RPT = N_PAD // NS

H2 = 128
RB = 1024


def _sc_mesh():
    return plsc.VectorSubcoreMesh(
        core_axis_name="c", subcore_axis_name="s",
        num_cores=NC, num_subcores=NS)


@functools.cache
def _build_deg_sc():
    return functools.partial(
        pl.kernel,
        out_type=jax.ShapeDtypeStruct((NC, N_PAD), jnp.float32),
        mesh=_sc_mesh(),
        scratch_types=[
            pltpu.VMEM((CPT, CHUNK), jnp.int32),
            pltpu.VMEM((CHUNK,), jnp.float32),
            pltpu.VMEM_SHARED((N_PAD,), jnp.float32),
        ],
    )(_deg_body)


def _deg_body(dst_hbm, ones_hbm, zeros_hbm, out_hbm, idx_v, ones_v, acc_sh):
    cid = lax.axis_index("c")
    sid = lax.axis_index("s")
    wid = sid * NC + cid
    pltpu.sync_copy(dst_hbm.at[wid], idx_v)
    pltpu.sync_copy(ones_hbm, ones_v)

    @pl.when(sid == 0)
    def _():
        pltpu.sync_copy(zeros_hbm.at[pl.ds(0, N_PAD)], acc_sh)

    plsc.subcore_barrier()

    def body(j, c):
        pltpu.sync_copy(ones_v, acc_sh.at[idx_v.at[j]], add=True)
        return c
    lax.fori_loop(0, CPT, body, 0)

    plsc.subcore_barrier()
    sl = pl.ds(sid * RPT, RPT)
    pltpu.sync_copy(acc_sh.at[sl], out_hbm.at[cid, sl])


@functools.cache
def _build_agg1_sc():
    return functools.partial(
        pl.kernel,
        out_type=jax.ShapeDtypeStruct((NC, N_PAD, H2), jnp.float32),
        mesh=_sc_mesh(),
        scratch_types=[
            pltpu.VMEM((CPT, CHUNK), jnp.int32),
            pltpu.VMEM((CPT, CHUNK), jnp.int32),
            pltpu.VMEM((CHUNK, H2), jnp.float32),
            pltpu.VMEM_SHARED((N_PAD, H2), jnp.float32),
        ],
    )(_agg1_body)


def _agg1_body(src_hbm, dst_hbm, p_hbm, zeros_hbm, out_hbm,
               src_v, dst_v, gbuf, acc_sh):
    cid = lax.axis_index("c")
    sid = lax.axis_index("s")
    wid = sid * NC + cid
    pltpu.sync_copy(src_hbm.at[wid], src_v)
    pltpu.sync_copy(dst_hbm.at[wid], dst_v)

    sl = pl.ds(sid * RPT, RPT)
    pltpu.sync_copy(zeros_hbm.at[sl], acc_sh.at[sl])
    plsc.subcore_barrier()

    def body(j, c):
        pltpu.sync_copy(p_hbm.at[src_v.at[j]], gbuf)
        pltpu.sync_copy(gbuf, acc_sh.at[dst_v.at[j]], add=True)
        return c
    lax.fori_loop(0, CPT, body, 0)

    plsc.subcore_barrier()
    pltpu.sync_copy(acc_sh.at[sl], out_hbm.at[cid, sl])


@functools.cache
def _build_agg2_sc():
    return functools.partial(
        pl.kernel,
        out_type=jax.ShapeDtypeStruct((NC, N_PAD), jnp.float32),
        mesh=_sc_mesh(),
        scratch_types=[
            pltpu.VMEM((CPT, CHUNK), jnp.int32),
            pltpu.VMEM((CPT, CHUNK), jnp.int32),
            pltpu.VMEM((CHUNK,), jnp.float32),
            pltpu.VMEM_SHARED((N_PAD,), jnp.float32),
        ],
    )(_agg2_body)


def _agg2_body(src_hbm, dst_hbm, q_hbm, zeros_hbm, out_hbm,
               src_v, dst_v, gbuf, acc_sh):
    cid = lax.axis_index("c")
    sid = lax.axis_index("s")
    wid = sid * NC + cid
    pltpu.sync_copy(src_hbm.at[wid], src_v)
    pltpu.sync_copy(dst_hbm.at[wid], dst_v)

    @pl.when(sid == 0)
    def _():
        pltpu.sync_copy(zeros_hbm.at[pl.ds(0, N_PAD)], acc_sh)

    plsc.subcore_barrier()

    def body(j, c):
        pltpu.sync_copy(q_hbm.at[src_v.at[j]], gbuf)
        pltpu.sync_copy(gbuf, acc_sh.at[dst_v.at[j]], add=True)
        return c
    lax.fori_loop(0, CPT, body, 0)

    plsc.subcore_barrier()
    sl = pl.ds(sid * RPT, RPT)
    pltpu.sync_copy(acc_sh.at[sl], out_hbm.at[cid, sl])


def _mm1_body(x_ref, w_ref, degp_ref, p_ref, dis_ref):
    h = jnp.dot(x_ref[...], w_ref[...], preferred_element_type=jnp.float32)
    deg = degp_ref[0] + degp_ref[1] + 2.0
    dis = lax.rsqrt(deg)
    ph = h * dis[:, None]
    p_ref[...] = jnp.concatenate(
        [ph, jnp.zeros((ph.shape[0], H2 - H), jnp.float32)], axis=1)
    dis_ref[...] = dis[:, None]


def _mm1(x_pad, W1, degp):
    return pl.pallas_call(
        _mm1_body,
        grid=(N_PAD // RB,),
        in_specs=[
            pl.BlockSpec((RB, D), lambda i: (i, 0)),
            pl.BlockSpec((D, H), lambda i: (0, 0)),
            pl.BlockSpec((NC, RB), lambda i: (0, i)),
        ],
        out_specs=[
            pl.BlockSpec((RB, H2), lambda i: (i, 0)),
            pl.BlockSpec((RB, 1), lambda i: (i, 0)),
        ],
        out_shape=[
            jax.ShapeDtypeStruct((N_PAD, H2), jnp.float32),
            jax.ShapeDtypeStruct((N_PAD, 1), jnp.float32),
        ],
    )(x_pad, W1, degp)


def _mid_body(s1_ref, p_ref, dis_ref, b1_ref, w2_ref, q_ref):
    s1 = s1_ref[0][:, :H] + s1_ref[1][:, :H]
    dis = dis_ref[...]
    r = jnp.maximum(dis * (s1 + 2.0 * p_ref[...][:, :H]) + b1_ref[...], 0.0)
    z = jnp.sum(r * w2_ref[...], axis=1, keepdims=True)
    q_ref[...] = dis * z


def _mid(s1p, p, dis, b1r, w2r):
    return pl.pallas_call(
        _mid_body,
        grid=(N_PAD // RB,),
        in_specs=[
            pl.BlockSpec((NC, RB, H2), lambda i: (0, i, 0)),
            pl.BlockSpec((RB, H2), lambda i: (i, 0)),
            pl.BlockSpec((RB, 1), lambda i: (i, 0)),
            pl.BlockSpec((1, H), lambda i: (0, 0)),
            pl.BlockSpec((1, H), lambda i: (0, 0)),
        ],
        out_specs=pl.BlockSpec((RB, 1), lambda i: (i, 0)),
        out_shape=jax.ShapeDtypeStruct((N_PAD, 1), jnp.float32),
    )(s1p, p, dis, b1r, w2r)


def _pool_body(s2_ref, q_ref, dis_ref, batch_ref, b2_ref, acc_ref, out_ref):
    i = pl.program_id(0)

    @pl.when(i == 0)
    def _():
        acc_ref[...] = jnp.zeros_like(acc_ref)

    s2 = s2_ref[0] + s2_ref[1]
    v = dis_ref[...][:, 0] * (s2 + 2.0 * q_ref[...][:, 0])
    gids = lax.broadcasted_iota(jnp.int32, (RB, G), 1)
    mask = batch_ref[...] == gids
    S = jnp.sum(jnp.where(mask, v[:, None], 0.0), axis=0)
    c = jnp.sum(mask.astype(jnp.float32), axis=0)
    acc_ref[...] += jnp.concatenate([S[None, :], c[None, :]], axis=0)

    @pl.when(i == pl.num_programs(0) - 1)
    def _():
        Sg = acc_ref[0, :]
        cg = acc_ref[1, :]
        pooled = (Sg + cg * b2_ref[0, 0]) / jnp.maximum(cg, 1.0)
        out_ref[...] = pooled[:, None]


def _pool(s2p, q, dis, batch_pad, b2r):
    _, out = pl.pallas_call(
        _pool_body,
        grid=(N_PAD // RB,),
        in_specs=[
            pl.BlockSpec((NC, RB), lambda i: (0, i)),
            pl.BlockSpec((RB, 1), lambda i: (i, 0)),
            pl.BlockSpec((RB, 1), lambda i: (i, 0)),
            pl.BlockSpec((RB, 1), lambda i: (i, 0)),
            pl.BlockSpec((1, 1), lambda i: (0, 0)),
        ],
        out_specs=[
            pl.BlockSpec((NC, G), lambda i: (0, 0)),
            pl.BlockSpec((G, 1), lambda i: (0, 0)),
        ],
        out_shape=[
            jax.ShapeDtypeStruct((NC, G), jnp.float32),
            jax.ShapeDtypeStruct((G, 1), jnp.float32),
        ],
    )(s2p, q, dis, batch_pad, b2r)
    return out


def kernel(x, edge_index, edge_attr, batch, W1, b1, W2, b2):
    del edge_attr
    src = edge_index[0]
    dst = edge_index[1]
    pad = E_PAD - E
    src3 = jnp.concatenate(
        [src, jnp.full((pad,), N, jnp.int32)]).reshape(NW, CPT, CHUNK)
    dst3 = jnp.concatenate(
        [dst, jnp.full((pad,), N, jnp.int32)]).reshape(NW, CPT, CHUNK)

    x_pad = jnp.pad(x, ((0, N_PAD - N), (0, 0)))
    batch_pad = jnp.pad(batch, (0, N_PAD - N),
                        constant_values=G).reshape(N_PAD, 1)
    zeros_big = jnp.zeros((N_PAD, H2), jnp.float32)
    zeros_flat = zeros_big.reshape(N_PAD * H2)
    ones_chunk = jnp.ones((CHUNK,), jnp.float32)

    degp = _build_deg_sc()(dst3, ones_chunk, zeros_flat)
    p, dis = _mm1(x_pad, W1, degp)
    s1p = _build_agg1_sc()(src3, dst3, p, zeros_big)
    q = _mid(s1p, p, dis, b1.reshape(1, H), W2.reshape(1, H))
    s2p = _build_agg2_sc()(src3, dst3, q.reshape(N_PAD), zeros_flat)
    return _pool(s2p, q, dis, batch_pad, b2.reshape(1, 1))


# speedup vs baseline: 11.4598x; 1.0702x over previous
import functools

import jax
import jax.numpy as jnp
from jax import lax
from jax.experimental import pallas as pl
from jax.experimental.pallas import tpu as pltpu
from jax.experimental.pallas import tpu_sc as plsc

N = 10000
E = 160000
D = 256
H = 64
G = 128

NC = 2
NS = 16
NW = NC * NS
CHUNK = 128
CPT = 40
E_PAD = NW * CPT * CHUNK
N_PAD = 10240
RPT = N_PAD // NS

H2 = 128
NBUF = 2
RB = 1024


def _sc_mesh():
    return plsc.VectorSubcoreMesh(
        core_axis_name="c", subcore_axis_name="s",
        num_cores=NC, num_subcores=NS)


@functools.cache
def _build_deg_sc():
    return functools.partial(
        pl.kernel,
        out_type=jax.ShapeDtypeStruct((NC, N_PAD), jnp.float32),
        mesh=_sc_mesh(),
        scratch_types=[
            pltpu.VMEM((CPT, CHUNK), jnp.int32),
            pltpu.VMEM((CHUNK,), jnp.float32),
            pltpu.VMEM_SHARED((N_PAD,), jnp.float32),
        ],
    )(_deg_body)


def _deg_body(dst_hbm, ones_hbm, zeros_hbm, out_hbm, idx_v, ones_v, acc_sh):
    cid = lax.axis_index("c")
    sid = lax.axis_index("s")
    wid = sid * NC + cid
    pltpu.sync_copy(dst_hbm.at[wid], idx_v)
    pltpu.sync_copy(ones_hbm, ones_v)

    @pl.when(sid == 0)
    def _():
        pltpu.sync_copy(zeros_hbm.at[pl.ds(0, N_PAD)], acc_sh)

    plsc.subcore_barrier()

    def body(j, c):
        pltpu.sync_copy(ones_v, acc_sh.at[idx_v.at[j]], add=True)
        return c
    lax.fori_loop(0, CPT, body, 0)

    plsc.subcore_barrier()
    sl = pl.ds(sid * RPT, RPT)
    pltpu.sync_copy(acc_sh.at[sl], out_hbm.at[cid, sl])


@functools.cache
def _build_agg1_sc():
    return functools.partial(
        pl.kernel,
        out_type=jax.ShapeDtypeStruct((NC, N_PAD, H2), jnp.float32),
        mesh=_sc_mesh(),
        scratch_types=[
            pltpu.VMEM((CPT, CHUNK), jnp.int32),
            pltpu.VMEM((CPT, CHUNK), jnp.int32),
        ]
        + [pltpu.VMEM((CHUNK, H2), jnp.float32) for _ in range(NBUF)]
        + [pltpu.SemaphoreType.DMA for _ in range(NBUF)]
        + [pltpu.VMEM_SHARED((N_PAD, H2), jnp.float32)],
    )(_agg1_body)


def _agg1_body(src_hbm, dst_hbm, p_hbm, zeros_hbm, out_hbm,
               src_v, dst_v, *rest):
    gbuf = rest[:NBUF]
    gsem = rest[NBUF:2 * NBUF]
    acc_sh = rest[2 * NBUF]
    cid = lax.axis_index("c")
    sid = lax.axis_index("s")
    wid = sid * NC + cid
    pltpu.sync_copy(src_hbm.at[wid], src_v)
    pltpu.sync_copy(dst_hbm.at[wid], dst_v)

    sl = pl.ds(sid * RPT, RPT)
    pltpu.sync_copy(zeros_hbm.at[sl], acc_sh.at[sl])
    plsc.subcore_barrier()

    descs = [
        pltpu.async_copy(p_hbm.at[src_v.at[b]], gbuf[b], gsem[b])
        for b in range(NBUF)
    ]

    def grp(g, c):
        for b in range(NBUF):
            j = g * NBUF + b
            descs[b].wait()
            pltpu.sync_copy(gbuf[b], acc_sh.at[dst_v.at[j]], add=True)
            pltpu.async_copy(p_hbm.at[src_v.at[j + NBUF]], gbuf[b], gsem[b])
        return c
    lax.fori_loop(0, CPT // NBUF - 1, grp, 0)

    g_last = CPT // NBUF - 1
    for b in range(NBUF):
        j = g_last * NBUF + b
        descs[b].wait()
        pltpu.sync_copy(gbuf[b], acc_sh.at[dst_v.at[j]], add=True)

    plsc.subcore_barrier()
    pltpu.sync_copy(acc_sh.at[sl], out_hbm.at[cid, sl])


@functools.cache
def _build_agg2_sc():
    return functools.partial(
        pl.kernel,
        out_type=jax.ShapeDtypeStruct((NC, N_PAD), jnp.float32),
        mesh=_sc_mesh(),
        scratch_types=[
            pltpu.VMEM((CPT, CHUNK), jnp.int32),
            pltpu.VMEM((CPT, CHUNK), jnp.int32),
            pltpu.VMEM((CHUNK,), jnp.float32),
            pltpu.VMEM_SHARED((N_PAD,), jnp.float32),
        ],
    )(_agg2_body)


def _agg2_body(src_hbm, dst_hbm, q_hbm, zeros_hbm, out_hbm,
               src_v, dst_v, gbuf, acc_sh):
    cid = lax.axis_index("c")
    sid = lax.axis_index("s")
    wid = sid * NC + cid
    pltpu.sync_copy(src_hbm.at[wid], src_v)
    pltpu.sync_copy(dst_hbm.at[wid], dst_v)

    @pl.when(sid == 0)
    def _():
        pltpu.sync_copy(zeros_hbm.at[pl.ds(0, N_PAD)], acc_sh)

    plsc.subcore_barrier()

    def body(j, c):
        pltpu.sync_copy(q_hbm.at[src_v.at[j]], gbuf)
        pltpu.sync_copy(gbuf, acc_sh.at[dst_v.at[j]], add=True)
        return c
    lax.fori_loop(0, CPT, body, 0)

    plsc.subcore_barrier()
    sl = pl.ds(sid * RPT, RPT)
    pltpu.sync_copy(acc_sh.at[sl], out_hbm.at[cid, sl])


def _mm1_body(x_ref, w_ref, degp_ref, p_ref, dis_ref):
    h = jnp.dot(x_ref[...], w_ref[...], preferred_element_type=jnp.float32)
    deg = degp_ref[0] + degp_ref[1] + 2.0
    dis = lax.rsqrt(deg)
    ph = h * dis[:, None]
    p_ref[...] = jnp.concatenate(
        [ph, jnp.zeros((ph.shape[0], H2 - H), jnp.float32)], axis=1)
    dis_ref[...] = dis[:, None]


def _mm1(x_pad, W1, degp):
    return pl.pallas_call(
        _mm1_body,
        grid=(N_PAD // RB,),
        in_specs=[
            pl.BlockSpec((RB, D), lambda i: (i, 0)),
            pl.BlockSpec((D, H), lambda i: (0, 0)),
            pl.BlockSpec((NC, RB), lambda i: (0, i)),
        ],
        out_specs=[
            pl.BlockSpec((RB, H2), lambda i: (i, 0)),
            pl.BlockSpec((RB, 1), lambda i: (i, 0)),
        ],
        out_shape=[
            jax.ShapeDtypeStruct((N_PAD, H2), jnp.float32),
            jax.ShapeDtypeStruct((N_PAD, 1), jnp.float32),
        ],
    )(x_pad, W1, degp)


def _mid_body(s1_ref, p_ref, dis_ref, b1_ref, w2_ref, q_ref):
    s1 = s1_ref[0][:, :H] + s1_ref[1][:, :H]
    dis = dis_ref[...]
    r = jnp.maximum(dis * (s1 + 2.0 * p_ref[...][:, :H]) + b1_ref[...], 0.0)
    z = jnp.sum(r * w2_ref[...], axis=1, keepdims=True)
    q_ref[...] = dis * z


def _mid(s1p, p, dis, b1r, w2r):
    return pl.pallas_call(
        _mid_body,
        grid=(N_PAD // RB,),
        in_specs=[
            pl.BlockSpec((NC, RB, H2), lambda i: (0, i, 0)),
            pl.BlockSpec((RB, H2), lambda i: (i, 0)),
            pl.BlockSpec((RB, 1), lambda i: (i, 0)),
            pl.BlockSpec((1, H), lambda i: (0, 0)),
            pl.BlockSpec((1, H), lambda i: (0, 0)),
        ],
        out_specs=pl.BlockSpec((RB, 1), lambda i: (i, 0)),
        out_shape=jax.ShapeDtypeStruct((N_PAD, 1), jnp.float32),
    )(s1p, p, dis, b1r, w2r)


def _pool_body(s2_ref, q_ref, dis_ref, batch_ref, b2_ref, acc_ref, out_ref):
    i = pl.program_id(0)

    @pl.when(i == 0)
    def _():
        acc_ref[...] = jnp.zeros_like(acc_ref)

    s2 = s2_ref[0] + s2_ref[1]
    v = dis_ref[...][:, 0] * (s2 + 2.0 * q_ref[...][:, 0])
    gids = lax.broadcasted_iota(jnp.int32, (RB, G), 1)
    mask = batch_ref[...] == gids
    S = jnp.sum(jnp.where(mask, v[:, None], 0.0), axis=0)
    c = jnp.sum(mask.astype(jnp.float32), axis=0)
    acc_ref[...] += jnp.concatenate([S[None, :], c[None, :]], axis=0)

    @pl.when(i == pl.num_programs(0) - 1)
    def _():
        Sg = acc_ref[0, :]
        cg = acc_ref[1, :]
        pooled = (Sg + cg * b2_ref[0, 0]) / jnp.maximum(cg, 1.0)
        out_ref[...] = pooled[:, None]


def _pool(s2p, q, dis, batch_pad, b2r):
    _, out = pl.pallas_call(
        _pool_body,
        grid=(N_PAD // RB,),
        in_specs=[
            pl.BlockSpec((NC, RB), lambda i: (0, i)),
            pl.BlockSpec((RB, 1), lambda i: (i, 0)),
            pl.BlockSpec((RB, 1), lambda i: (i, 0)),
            pl.BlockSpec((RB, 1), lambda i: (i, 0)),
            pl.BlockSpec((1, 1), lambda i: (0, 0)),
        ],
        out_specs=[
            pl.BlockSpec((NC, G), lambda i: (0, 0)),
            pl.BlockSpec((G, 1), lambda i: (0, 0)),
        ],
        out_shape=[
            jax.ShapeDtypeStruct((NC, G), jnp.float32),
            jax.ShapeDtypeStruct((G, 1), jnp.float32),
        ],
    )(s2p, q, dis, batch_pad, b2r)
    return out


def kernel(x, edge_index, edge_attr, batch, W1, b1, W2, b2):
    del edge_attr
    src = edge_index[0]
    dst = edge_index[1]
    pad = E_PAD - E
    src3 = jnp.concatenate(
        [src, jnp.full((pad,), N, jnp.int32)]).reshape(NW, CPT, CHUNK)
    dst3 = jnp.concatenate(
        [dst, jnp.full((pad,), N, jnp.int32)]).reshape(NW, CPT, CHUNK)

    x_pad = jnp.pad(x, ((0, N_PAD - N), (0, 0)))
    batch_pad = jnp.pad(batch, (0, N_PAD - N),
                        constant_values=G).reshape(N_PAD, 1)
    zeros_big = jnp.zeros((N_PAD, H2), jnp.float32)
    zeros_flat = zeros_big.reshape(N_PAD * H2)
    ones_chunk = jnp.ones((CHUNK,), jnp.float32)

    degp = _build_deg_sc()(dst3, ones_chunk, zeros_flat)
    p, dis = _mm1(x_pad, W1, degp)
    s1p = _build_agg1_sc()(src3, dst3, p, zeros_big)
    q = _mid(s1p, p, dis, b1.reshape(1, H), W2.reshape(1, H))
    s2p = _build_agg2_sc()(src3, dst3, q.reshape(N_PAD), zeros_flat)
    return _pool(s2p, q, dis, batch_pad, b2.reshape(1, 1))
